# Initial kernel scaffold; baseline (speedup 1.0000x reference)
import jax
import jax.numpy as jnp
from jax.experimental import pallas as pl


def kernel(x, edge_index, Wp, bp, Wa1, ba1, Wa2, ba2, Wih, Whh, bih, bhh, Wo, bo):
    raise NotImplementedError("write your pallas kernel here")


# speedup vs baseline: 2.0601x; 2.0601x over previous
import functools

import jax
import jax.numpy as jnp
from jax import lax
from jax.experimental import pallas as pl
from jax.experimental.pallas import tpu as pltpu
from jax.experimental.pallas import tpu_sc as plsc

N = 10000
E = 320000
H = 128
ROUNDS = 5

NC, NS, L = 2, 16, 16
NW = NC * NS
CH = 128
SB = 16
NSC = 5
NCHUNK = NSC * SB
EPW = NCHUNK * CH
EPAD = NW * EPW
RPT = N // NS

_MESH = plsc.VectorSubcoreMesh(core_axis_name="c", subcore_axis_name="s")


def _hsum(v):
    return plsc.cumsum(v)[15]


def _hmax(v):
    return plsc.cummax(v)[15]


def _pre_body(x_ref, wpt, bp, wlt, wrt, ba1, h_ref, a_ref, b_ref):
    h = jnp.maximum(jnp.dot(x_ref[...], wpt[...],
                            preferred_element_type=jnp.float32) + bp[...], 0.0)
    h_ref[...] = h
    a_ref[...] = jnp.dot(h, wlt[...], preferred_element_type=jnp.float32)
    b_ref[...] = jnp.dot(h, wrt[...],
                         preferred_element_type=jnp.float32) + ba1[...]


def _gru(h, num_ref, den_ref, wiht, whht, bih, bhh):
    num = num_ref[0] + num_ref[1]
    den = jnp.sum(den_ref[0] + den_ref[1], axis=-1, keepdims=True)
    agg = num / (den + 1e-8)
    gi = jnp.dot(agg, wiht[...], preferred_element_type=jnp.float32) + bih[...]
    gh = jnp.dot(h, whht[...], preferred_element_type=jnp.float32) + bhh[...]
    r = jax.nn.sigmoid(gi[:, 0:H] + gh[:, 0:H])
    z = jax.nn.sigmoid(gi[:, H:2 * H] + gh[:, H:2 * H])
    n = jnp.tanh(gi[:, 2 * H:] + r * gh[:, 2 * H:])
    return (1.0 - z) * n + z * h


def _step_body(h_ref, num_ref, den_ref, wiht, whht, bih, bhh, wlt, wrt, ba1,
               h_out, a_out, b_out):
    hn = _gru(h_ref[...], num_ref, den_ref, wiht, whht, bih, bhh)
    h_out[...] = hn
    a_out[...] = jnp.dot(hn, wlt[...], preferred_element_type=jnp.float32)
    b_out[...] = jnp.dot(hn, wrt[...],
                         preferred_element_type=jnp.float32) + ba1[...]


def _final_body(h_ref, num_ref, den_ref, wiht, whht, bih, bhh, wot, bo,
                out_ref):
    hn = _gru(h_ref[...], num_ref, den_ref, wiht, whht, bih, bhh)
    out_ref[...] = jnp.dot(hn, wot[...],
                           preferred_element_type=jnp.float32) + bo[...]


_BLK = 1000
_GRID = N // _BLK


def _row_spec():
    return pl.BlockSpec((_BLK, H), lambda i: (i, 0))


def _whole(shape):
    return pl.BlockSpec(shape, lambda i: tuple(0 for _ in shape))


def _tc_pre(x, wpt, bp, wlt, wrt, ba1):
    return pl.pallas_call(
        _pre_body,
        grid=(_GRID,),
        in_specs=[_row_spec(), _whole((H, H)), _whole((1, H)),
                  _whole((H, H)), _whole((H, H)), _whole((1, H))],
        out_specs=[_row_spec(), _row_spec(), _row_spec()],
        out_shape=[jax.ShapeDtypeStruct((N, H), jnp.float32)] * 3,
    )(x, wpt, bp, wlt, wrt, ba1)


def _agg_specs():
    return [pl.BlockSpec((2, _BLK, H), lambda i: (0, i, 0)),
            pl.BlockSpec((2, _BLK, L), lambda i: (0, i, 0))]


def _tc_step(h, num, den, wiht, whht, bih, bhh, wlt, wrt, ba1):
    return pl.pallas_call(
        _step_body,
        grid=(_GRID,),
        in_specs=[_row_spec()] + _agg_specs() +
                 [_whole((H, 3 * H)), _whole((H, 3 * H)), _whole((1, 3 * H)),
                  _whole((1, 3 * H)), _whole((H, H)), _whole((H, H)),
                  _whole((1, H))],
        out_specs=[_row_spec(), _row_spec(), _row_spec()],
        out_shape=[jax.ShapeDtypeStruct((N, H), jnp.float32)] * 3,
    )(h, num, den, wiht, whht, bih, bhh, wlt, wrt, ba1)


def _tc_final(h, num, den, wiht, whht, bih, bhh, wot, bo):
    return pl.pallas_call(
        _final_body,
        grid=(_GRID,),
        in_specs=[_row_spec()] + _agg_specs() +
                 [_whole((H, 3 * H)), _whole((H, 3 * H)), _whole((1, 3 * H)),
                  _whole((1, 3 * H)), _whole((H, H)), _whole((1, H))],
        out_specs=[_row_spec()],
        out_shape=[jax.ShapeDtypeStruct((N, H), jnp.float32)],
    )(h, num, den, wiht, whht, bih, bhh, wot, bo)[0]


@functools.partial(
    pl.kernel,
    out_type=[jax.ShapeDtypeStruct((NW, NCHUNK, CH), jnp.float32),
              jax.ShapeDtypeStruct((NW, L), jnp.float32)],
    mesh=_MESH,
    compiler_params=pltpu.CompilerParams(needs_layout_passes=False, use_tc_tiling_on_sc=False),
    scratch_types=[
        pltpu.VMEM((NCHUNK, CH), jnp.int32),
        pltpu.VMEM((NCHUNK, CH), jnp.int32),
        pltpu.VMEM((CH, H), jnp.float32),
        pltpu.VMEM((CH, H), jnp.float32),
        pltpu.VMEM((CH,), jnp.float32),
        pltpu.VMEM((H,), jnp.float32),
        pltpu.SemaphoreType.DMA,
        pltpu.SemaphoreType.DMA,
    ],
)
def _sc_scores(a_hbm, b_hbm, wa2_hbm, row_hbm, col_hbm, s_hbm, pmax_hbm,
               row_v, col_v, arows, brows, sbuf, wa2_v, sem_a, sem_b):
    wid = lax.axis_index("s") * NC + lax.axis_index("c")
    pltpu.sync_copy(row_hbm.at[wid], row_v)
    pltpu.sync_copy(col_hbm.at[wid], col_v)
    pltpu.sync_copy(wa2_hbm, wa2_v)
    wvecs = [wa2_v[pl.ds(16 * k, 16)] for k in range(8)]
    lanes = lax.iota(jnp.int32, 16)
    masks = [lanes == l for l in range(16)]

    def chunk(j, smax):
        ca = pltpu.async_copy(a_hbm.at[row_v.at[j]], arows, sem_a)
        cb = pltpu.async_copy(b_hbm.at[col_v.at[j]], brows, sem_b)
        ca.wait()
        cb.wait()

        def group(g, smax):
            svec = jnp.zeros((16,), jnp.float32)
            for l in range(16):
                e = g * 16 + l
                acc = jnp.zeros((16,), jnp.float32)
                for k in range(8):
                    va = arows[e, pl.ds(16 * k, 16)]
                    vb = brows[e, pl.ds(16 * k, 16)]
                    acc = acc + jnp.maximum(va + vb, 0.0) * wvecs[k]
                sval = _hsum(acc)
                eid = wid * EPW + j * CH + e
                sval = jnp.where(eid < E, sval, jnp.float32(-1e30))
                svec = jnp.where(masks[l], sval, svec)
            sbuf[pl.ds(g * 16, 16)] = svec
            return jnp.maximum(smax, _hmax(svec))

        smax = lax.fori_loop(0, CH // 16, group, smax)
        pltpu.sync_copy(sbuf, s_hbm.at[wid, j])
        return smax

    smax = lax.fori_loop(0, NCHUNK, chunk, jnp.float32(-1e30))
    sbuf[pl.ds(0, 16)] = jnp.broadcast_to(smax, (16,))
    pltpu.sync_copy(sbuf.at[pl.ds(0, 16)], pmax_hbm.at[wid])


@functools.partial(
    pl.kernel,
    out_type=[jax.ShapeDtypeStruct((NC, N, H), jnp.float32),
              jax.ShapeDtypeStruct((NC, N, L), jnp.float32)],
    mesh=_MESH,
    compiler_params=pltpu.CompilerParams(needs_layout_passes=False, use_tc_tiling_on_sc=False),
    scratch_types=[
        pltpu.VMEM((SB, CH), jnp.int32),
        pltpu.VMEM((SB, CH), jnp.int32),
        pltpu.VMEM((CH, H), jnp.float32),
        pltpu.VMEM((CH, H), jnp.float32),
        pltpu.VMEM((CH, L), jnp.float32),
        pltpu.VMEM((CH,), jnp.float32),
        pltpu.VMEM((NW, L), jnp.float32),
        pltpu.VMEM_SHARED((N, H), jnp.float32),
        pltpu.VMEM_SHARED((N, L), jnp.float32),
        pltpu.SemaphoreType.DMA,
        pltpu.SemaphoreType.DMA,
    ],
)
def _sc_aggregate(h_hbm, s_hbm, pmax_hbm, row_hbm, col_hbm, num_hbm, den_hbm,
                  row_v, col_v, hi, vbuf, dbuf, wbuf, pmax_v,
                  acc_num, acc_den, sem_a, sem_b):
    c = lax.axis_index("c")
    sid = lax.axis_index("s")
    wid = sid * NC + c
    pltpu.sync_copy(pmax_hbm, pmax_v)

    mv = pmax_v[0]
    for r in range(1, NW):
        mv = jnp.maximum(mv, pmax_v[r])
    gmax = _hmax(mv)

    zero16 = jnp.zeros((16,), jnp.float32)
    mask0 = lax.iota(jnp.int32, 16) == 0

    def zrow(r, _):
        for k in range(8):
            vbuf[r, pl.ds(16 * k, 16)] = zero16
        dbuf[r, pl.ds(0, 16)] = zero16
        return 0

    lax.fori_loop(0, CH, zrow, 0)
    tbase = sid * RPT
    for i in range(5):
        pltpu.sync_copy(vbuf.at[pl.ds(0, 125)],
                        acc_num.at[pl.ds(tbase + i * 125, 125)])
        pltpu.sync_copy(dbuf.at[pl.ds(0, 125)],
                        acc_den.at[pl.ds(tbase + i * 125, 125)])
    plsc.subcore_barrier()

    def superchunk(sc, _):
        pltpu.sync_copy(row_hbm.at[wid, pl.ds(sc * SB, SB)], row_v)
        pltpu.sync_copy(col_hbm.at[wid, pl.ds(sc * SB, SB)], col_v)

        def chunk(jj, _):
            pltpu.sync_copy(s_hbm.at[wid, sc * SB + jj], wbuf)
            ca = pltpu.async_copy(h_hbm.at[row_v.at[jj]], hi, sem_a)
            cb = pltpu.async_copy(h_hbm.at[col_v.at[jj]], vbuf, sem_b)
            ca.wait()
            cb.wait()

            def group(g, _):
                wv = jnp.exp(wbuf[pl.ds(g * 16, 16)] - gmax)
                for l in range(16):
                    e = g * 16 + l
                    ws = wv[l]
                    dbuf[e, pl.ds(0, 16)] = jnp.where(mask0, ws, zero16)
                    for k in range(8):
                        d = jnp.abs(hi[e, pl.ds(16 * k, 16)]
                                    - vbuf[e, pl.ds(16 * k, 16)])
                        vbuf[e, pl.ds(16 * k, 16)] = d * ws
                return 0

            lax.fori_loop(0, CH // 16, group, 0)
            pltpu.sync_copy(vbuf, acc_num.at[row_v.at[jj]], add=True)
            pltpu.sync_copy(dbuf, acc_den.at[row_v.at[jj]], add=True)
            return 0

        lax.fori_loop(0, SB, chunk, 0)
        return 0

    lax.fori_loop(0, NSC, superchunk, 0)
    plsc.subcore_barrier()
    pltpu.sync_copy(acc_num.at[pl.ds(tbase, RPT)],
                    num_hbm.at[c, pl.ds(tbase, RPT)])
    pltpu.sync_copy(acc_den.at[pl.ds(tbase, RPT)],
                    den_hbm.at[c, pl.ds(tbase, RPT)])


def kernel(x, edge_index, Wp, bp, Wa1, ba1, Wa2, ba2, Wih, Whh, bih, bhh,
           Wo, bo):
    del ba2
    row = edge_index[0].astype(jnp.int32)
    col = edge_index[1].astype(jnp.int32)
    pad = jnp.zeros((EPAD - E,), jnp.int32)
    rowp = jnp.concatenate([row, pad]).reshape(NW, NCHUNK, CH)
    colp = jnp.concatenate([col, pad]).reshape(NW, NCHUNK, CH)

    wpt = Wp.T
    wlt = Wa1[:, :H].T
    wrt = Wa1[:, H:].T
    wa2v = Wa2.reshape(H)
    wiht = Wih.T
    whht = Whh.T
    wot = Wo.T
    bp2 = bp.reshape(1, H)
    ba12 = ba1.reshape(1, H)
    bih2 = bih.reshape(1, 3 * H)
    bhh2 = bhh.reshape(1, 3 * H)
    bo2 = bo.reshape(1, H)

    h, a, b = _tc_pre(x, wpt, bp2, wlt, wrt, ba12)
    for r in range(ROUNDS):
        s, pmax = _sc_scores(a, b, wa2v, rowp, colp)
        num, den = _sc_aggregate(h, s, pmax, rowp, colp)
        if r < ROUNDS - 1:
            h, a, b = _tc_step(h, num, den, wiht, whht, bih2, bhh2,
                               wlt, wrt, ba12)
        else:
            out = _tc_final(h, num, den, wiht, whht, bih2, bhh2, wot, bo2)
    return out


# speedup vs baseline: 2.3890x; 1.1596x over previous
import functools

import jax
import jax.numpy as jnp
from jax import lax
from jax.experimental import pallas as pl
from jax.experimental.pallas import tpu as pltpu
from jax.experimental.pallas import tpu_sc as plsc

N = 10000
E = 320000
H = 128
ROUNDS = 5

NC, NS, L = 2, 16, 16
NW = NC * NS
CH = 128
SB = 16
NSC = 5
NCHUNK = NSC * SB
EPW = NCHUNK * CH
EPAD = NW * EPW
RPT = N // NS

_MESH = plsc.VectorSubcoreMesh(core_axis_name="c", subcore_axis_name="s")


def _hsum(v):
    return plsc.cumsum(v)[15]


def _hmax(v):
    return plsc.cummax(v)[15]


def _pre_body(x_ref, wpt, bp, wlt, wrt, ba1, h_ref, a_ref, b_ref):
    h = jnp.maximum(jnp.dot(x_ref[...], wpt[...],
                            preferred_element_type=jnp.float32) + bp[...], 0.0)
    h_ref[...] = h
    a_ref[...] = jnp.dot(h, wlt[...], preferred_element_type=jnp.float32)
    b_ref[...] = jnp.dot(h, wrt[...],
                         preferred_element_type=jnp.float32) + ba1[...]


def _gru(h, num_ref, den_ref, wiht, whht, bih, bhh):
    num = num_ref[0] + num_ref[1]
    den = jnp.sum(den_ref[0] + den_ref[1], axis=-1, keepdims=True)
    agg = num / (den + 1e-8)
    gi = jnp.dot(agg, wiht[...], preferred_element_type=jnp.float32) + bih[...]
    gh = jnp.dot(h, whht[...], preferred_element_type=jnp.float32) + bhh[...]
    r = jax.nn.sigmoid(gi[:, 0:H] + gh[:, 0:H])
    z = jax.nn.sigmoid(gi[:, H:2 * H] + gh[:, H:2 * H])
    n = jnp.tanh(gi[:, 2 * H:] + r * gh[:, 2 * H:])
    return (1.0 - z) * n + z * h


def _step_body(h_ref, num_ref, den_ref, wiht, whht, bih, bhh, wlt, wrt, ba1,
               h_out, a_out, b_out):
    hn = _gru(h_ref[...], num_ref, den_ref, wiht, whht, bih, bhh)
    h_out[...] = hn
    a_out[...] = jnp.dot(hn, wlt[...], preferred_element_type=jnp.float32)
    b_out[...] = jnp.dot(hn, wrt[...],
                         preferred_element_type=jnp.float32) + ba1[...]


def _final_body(h_ref, num_ref, den_ref, wiht, whht, bih, bhh, wot, bo,
                out_ref):
    hn = _gru(h_ref[...], num_ref, den_ref, wiht, whht, bih, bhh)
    out_ref[...] = jnp.dot(hn, wot[...],
                           preferred_element_type=jnp.float32) + bo[...]


_BLK = 1000
_GRID = N // _BLK


def _row_spec():
    return pl.BlockSpec((_BLK, H), lambda i: (i, 0))


def _whole(shape):
    return pl.BlockSpec(shape, lambda i: tuple(0 for _ in shape))


def _tc_pre(x, wpt, bp, wlt, wrt, ba1):
    return pl.pallas_call(
        _pre_body,
        grid=(_GRID,),
        in_specs=[_row_spec(), _whole((H, H)), _whole((1, H)),
                  _whole((H, H)), _whole((H, H)), _whole((1, H))],
        out_specs=[_row_spec(), _row_spec(), _row_spec()],
        out_shape=[jax.ShapeDtypeStruct((N, H), jnp.float32)] * 3,
    )(x, wpt, bp, wlt, wrt, ba1)


def _agg_specs():
    return [pl.BlockSpec((2, _BLK, H), lambda i: (0, i, 0)),
            pl.BlockSpec((2, _BLK, L), lambda i: (0, i, 0))]


def _tc_step(h, num, den, wiht, whht, bih, bhh, wlt, wrt, ba1):
    return pl.pallas_call(
        _step_body,
        grid=(_GRID,),
        in_specs=[_row_spec()] + _agg_specs() +
                 [_whole((H, 3 * H)), _whole((H, 3 * H)), _whole((1, 3 * H)),
                  _whole((1, 3 * H)), _whole((H, H)), _whole((H, H)),
                  _whole((1, H))],
        out_specs=[_row_spec(), _row_spec(), _row_spec()],
        out_shape=[jax.ShapeDtypeStruct((N, H), jnp.float32)] * 3,
    )(h, num, den, wiht, whht, bih, bhh, wlt, wrt, ba1)


def _tc_final(h, num, den, wiht, whht, bih, bhh, wot, bo):
    return pl.pallas_call(
        _final_body,
        grid=(_GRID,),
        in_specs=[_row_spec()] + _agg_specs() +
                 [_whole((H, 3 * H)), _whole((H, 3 * H)), _whole((1, 3 * H)),
                  _whole((1, 3 * H)), _whole((H, H)), _whole((1, H))],
        out_specs=[_row_spec()],
        out_shape=[jax.ShapeDtypeStruct((N, H), jnp.float32)],
    )(h, num, den, wiht, whht, bih, bhh, wot, bo)[0]


@functools.partial(
    pl.kernel,
    out_type=[jax.ShapeDtypeStruct((NW, NCHUNK, CH), jnp.float32),
              jax.ShapeDtypeStruct((NW, L), jnp.float32)],
    mesh=_MESH,
    compiler_params=pltpu.CompilerParams(needs_layout_passes=False, use_tc_tiling_on_sc=False),
    scratch_types=[
        pltpu.VMEM((NCHUNK, CH), jnp.int32),
        pltpu.VMEM((NCHUNK, CH), jnp.int32),
        pltpu.VMEM((CH, H), jnp.float32),
        pltpu.VMEM((CH, H), jnp.float32),
        pltpu.VMEM((CH, H), jnp.float32),
        pltpu.VMEM((CH, H), jnp.float32),
        pltpu.VMEM((8, CH), jnp.float32),
        pltpu.VMEM((H,), jnp.float32),
        pltpu.SemaphoreType.DMA,
        pltpu.SemaphoreType.DMA,
        pltpu.SemaphoreType.DMA,
        pltpu.SemaphoreType.DMA,
    ],
)
def _sc_scores(a_hbm, b_hbm, wa2_hbm, row_hbm, col_hbm, s_hbm, pmax_hbm,
               row_v, col_v, ar0, br0, ar1, br1, sbuf, wa2_v,
               sa0, sb0, sa1, sb1):
    wid = lax.axis_index("s") * NC + lax.axis_index("c")
    pltpu.sync_copy(row_hbm.at[wid], row_v)
    pltpu.sync_copy(col_hbm.at[wid], col_v)
    pltpu.sync_copy(wa2_hbm, wa2_v)
    wvecs = [wa2_v[pl.ds(16 * k, 16)] for k in range(8)]
    lanes = lax.iota(jnp.int32, 16)
    masks = [lanes == l for l in range(16)]
    slots = ((ar0, br0, sa0, sb0), (ar1, br1, sa1, sb1))

    def issue(j, slot):
        ar, br, sa, sb = slots[slot]
        pltpu.async_copy(a_hbm.at[row_v.at[j]], ar, sa)
        pltpu.async_copy(b_hbm.at[col_v.at[j]], br, sb)

    def wait(slot):
        ar, br, sa, sb = slots[slot]
        pltpu.make_async_copy(a_hbm.at[row_v.at[0]], ar, sa).wait()
        pltpu.make_async_copy(b_hbm.at[col_v.at[0]], br, sb).wait()

    def compute(j, slot, smax):
        ar, br, _, _ = slots[slot]
        jrow = lax.rem(j, 8)

        def group(g, smax):
            svec = jnp.zeros((16,), jnp.float32)
            for l in range(16):
                e = g * 16 + l
                acc = jnp.zeros((16,), jnp.float32)
                for k in range(8):
                    va = ar[e, pl.ds(16 * k, 16)]
                    vb = br[e, pl.ds(16 * k, 16)]
                    acc = acc + jnp.maximum(va + vb, 0.0) * wvecs[k]
                sval = _hsum(acc)
                eid = wid * EPW + j * CH + e
                sval = jnp.where(eid < E, sval, jnp.float32(-1e30))
                svec = jnp.where(masks[l], sval, svec)
            sbuf[jrow, pl.ds(g * 16, 16)] = svec
            return jnp.maximum(smax, _hmax(svec))

        return lax.fori_loop(0, CH // 16, group, smax)

    issue(0, 0)
    issue(1, 1)

    def pair(p, smax):
        j = 2 * p
        wait(0)
        smax = compute(j, 0, smax)

        @pl.when(j + 2 < NCHUNK)
        def _():
            issue(j + 2, 0)

        wait(1)
        smax = compute(j + 1, 1, smax)

        @pl.when(j + 3 < NCHUNK)
        def _():
            issue(j + 3, 1)

        @pl.when(lax.rem(j + 1, 8) == 7)
        def _():
            pltpu.sync_copy(sbuf, s_hbm.at[wid, pl.ds(j - 6, 8)])

        return smax

    smax = lax.fori_loop(0, NCHUNK // 2, pair, jnp.float32(-1e30))
    sbuf[0, pl.ds(0, 16)] = jnp.broadcast_to(smax, (16,))
    pltpu.sync_copy(sbuf.at[0, pl.ds(0, 16)], pmax_hbm.at[wid])


CH2 = 64
NCH2 = EPW // CH2
SB2 = 16
NSC2 = NCH2 // SB2


@functools.partial(
    pl.kernel,
    out_type=[jax.ShapeDtypeStruct((NC, N, H), jnp.float32),
              jax.ShapeDtypeStruct((NC, N, L), jnp.float32)],
    mesh=_MESH,
    compiler_params=pltpu.CompilerParams(needs_layout_passes=False, use_tc_tiling_on_sc=False),
    scratch_types=[
        pltpu.VMEM((SB2, CH2), jnp.int32),
        pltpu.VMEM((SB2, CH2), jnp.int32),
        pltpu.VMEM((CH2, H), jnp.float32),
        pltpu.VMEM((CH2, H), jnp.float32),
        pltpu.VMEM((CH2, H), jnp.float32),
        pltpu.VMEM((CH2, H), jnp.float32),
        pltpu.VMEM((CH2, L), jnp.float32),
        pltpu.VMEM((CH2, L), jnp.float32),
        pltpu.VMEM((SB2 * CH2,), jnp.float32),
        pltpu.VMEM((NW, L), jnp.float32),
        pltpu.VMEM_SHARED((N, H), jnp.float32),
        pltpu.VMEM_SHARED((N, L), jnp.float32),
        pltpu.SemaphoreType.DMA,
        pltpu.SemaphoreType.DMA,
        pltpu.SemaphoreType.DMA,
        pltpu.SemaphoreType.DMA,
        pltpu.SemaphoreType.DMA,
        pltpu.SemaphoreType.DMA,
        pltpu.SemaphoreType.DMA,
        pltpu.SemaphoreType.DMA,
        pltpu.SemaphoreType.DMA,
        pltpu.SemaphoreType.DMA,
        pltpu.SemaphoreType.DMA,
    ],
)
def _sc_aggregate(h_hbm, s_hbm, pmax_hbm, row_hbm, col_hbm, num_hbm, den_hbm,
                  row_v, col_v, hi0, vb0, hi1, vb1, db0, db1, wsbuf, pmax_v,
                  acc_num, acc_den,
                  sa0, sb0, sa1, sb1, sn0, sd0, sn1, sd1, si0, si1, si2):
    c = lax.axis_index("c")
    sid = lax.axis_index("s")
    wid = sid * NC + c
    pltpu.sync_copy(pmax_hbm, pmax_v)

    mv = pmax_v[0]
    for r in range(1, NW):
        mv = jnp.maximum(mv, pmax_v[r])
    gmax = _hmax(mv)

    zero16 = jnp.zeros((16,), jnp.float32)
    mask0 = lax.iota(jnp.int32, 16) == 0

    def zrow(r, _):
        for k in range(8):
            vb0[r, pl.ds(16 * k, 16)] = zero16
        db0[r, pl.ds(0, 16)] = zero16
        return 0

    lax.fori_loop(0, CH2, zrow, 0)
    tbase = sid * RPT
    for i in range(9):
        pltpu.sync_copy(vb0, acc_num.at[pl.ds(tbase + i * CH2, CH2)])
        pltpu.sync_copy(db0, acc_den.at[pl.ds(tbase + i * CH2, CH2)])
    pltpu.sync_copy(vb0.at[pl.ds(0, 49)],
                    acc_num.at[pl.ds(tbase + 9 * CH2, 49)])
    pltpu.sync_copy(db0.at[pl.ds(0, 49)],
                    acc_den.at[pl.ds(tbase + 9 * CH2, 49)])
    plsc.subcore_barrier()

    slots = ((hi0, vb0, db0, sa0, sb0, sn0, sd0),
             (hi1, vb1, db1, sa1, sb1, sn1, sd1))

    def issue_gather(jj, slot):
        hi, vb, _, sa, sb, _, _ = slots[slot]
        pltpu.async_copy(h_hbm.at[row_v.at[jj]], hi, sa)
        pltpu.async_copy(h_hbm.at[col_v.at[jj]], vb, sb)

    def wait_gather(slot):
        hi, vb, _, sa, sb, _, _ = slots[slot]
        pltpu.make_async_copy(h_hbm.at[row_v.at[0]], hi, sa).wait()
        pltpu.make_async_copy(h_hbm.at[col_v.at[0]], vb, sb).wait()

    def issue_scatter(jj, slot):
        _, vb, db, _, _, sn, sd = slots[slot]
        pltpu.async_copy(vb, acc_num.at[row_v.at[jj]], sn, add=True)
        pltpu.async_copy(db, acc_den.at[row_v.at[jj]], sd, add=True)

    def wait_scatter(slot):
        _, vb, db, _, _, sn, sd = slots[slot]
        pltpu.make_async_copy(vb, acc_num.at[row_v.at[0]], sn).wait()
        pltpu.make_async_copy(db, acc_den.at[row_v.at[0]], sd).wait()

    def compute(jj, slot):
        hi, vb, db, _, _, _, _ = slots[slot]

        def group(g, _):
            wv = jnp.exp(wsbuf[pl.ds(jj * CH2 + g * 16, 16)] - gmax)
            for l in range(16):
                e = g * 16 + l
                ws = wv[l]
                db[e, pl.ds(0, 16)] = jnp.where(mask0, ws, zero16)
                for k in range(8):
                    d = jnp.abs(hi[e, pl.ds(16 * k, 16)]
                                - vb[e, pl.ds(16 * k, 16)])
                    vb[e, pl.ds(16 * k, 16)] = d * ws
            return 0

        lax.fori_loop(0, CH2 // 16, group, 0)

    def superchunk(sc, _):
        ci = pltpu.async_copy(row_hbm.at[wid, pl.ds(sc * SB2, SB2)],
                              row_v, si0)
        cj = pltpu.async_copy(col_hbm.at[wid, pl.ds(sc * SB2, SB2)],
                              col_v, si1)
        cs = pltpu.async_copy(s_hbm.at[wid, pl.ds(sc * SB2 * CH2, SB2 * CH2)],
                              wsbuf, si2)
        ci.wait()
        cj.wait()
        cs.wait()
        issue_gather(0, 0)
        issue_gather(1, 1)

        def pair(p, _):
            jj = 2 * p
            wait_gather(0)
            compute(jj, 0)
            issue_scatter(jj, 0)
            wait_gather(1)
            compute(jj + 1, 1)
            issue_scatter(jj + 1, 1)
            wait_scatter(0)

            @pl.when(jj + 2 < SB2)
            def _():
                issue_gather(jj + 2, 0)

            wait_scatter(1)

            @pl.when(jj + 3 < SB2)
            def _():
                issue_gather(jj + 3, 1)

            return 0

        lax.fori_loop(0, SB2 // 2, pair, 0)
        return 0

    lax.fori_loop(0, NSC2, superchunk, 0)
    plsc.subcore_barrier()
    pltpu.sync_copy(acc_num.at[pl.ds(tbase, RPT)],
                    num_hbm.at[c, pl.ds(tbase, RPT)])
    pltpu.sync_copy(acc_den.at[pl.ds(tbase, RPT)],
                    den_hbm.at[c, pl.ds(tbase, RPT)])


def kernel(x, edge_index, Wp, bp, Wa1, ba1, Wa2, ba2, Wih, Whh, bih, bhh,
           Wo, bo):
    del ba2
    row = edge_index[0].astype(jnp.int32)
    col = edge_index[1].astype(jnp.int32)
    pad = jnp.zeros((EPAD - E,), jnp.int32)
    rowf = jnp.concatenate([row, pad])
    colf = jnp.concatenate([col, pad])
    rowp = rowf.reshape(NW, NCHUNK, CH)
    colp = colf.reshape(NW, NCHUNK, CH)
    rowp2 = rowf.reshape(NW, NCH2, CH2)
    colp2 = colf.reshape(NW, NCH2, CH2)

    wpt = Wp.T
    wlt = Wa1[:, :H].T
    wrt = Wa1[:, H:].T
    wa2v = Wa2.reshape(H)
    wiht = Wih.T
    whht = Whh.T
    wot = Wo.T
    bp2 = bp.reshape(1, H)
    ba12 = ba1.reshape(1, H)
    bih2 = bih.reshape(1, 3 * H)
    bhh2 = bhh.reshape(1, 3 * H)
    bo2 = bo.reshape(1, H)

    h, a, b = _tc_pre(x, wpt, bp2, wlt, wrt, ba12)
    for r in range(ROUNDS):
        s, pmax = _sc_scores(a, b, wa2v, rowp, colp)
        num, den = _sc_aggregate(h, s.reshape(NW, EPW), pmax, rowp2, colp2)
        if r < ROUNDS - 1:
            h, a, b = _tc_step(h, num, den, wiht, whht, bih2, bhh2,
                               wlt, wrt, ba12)
        else:
            out = _tc_final(h, num, den, wiht, whht, bih2, bhh2, wot, bo2)
    return out


# speedup vs baseline: 2.7863x; 1.1663x over previous
import functools

import jax
import jax.numpy as jnp
from jax import lax
from jax.experimental import pallas as pl
from jax.experimental.pallas import tpu as pltpu
from jax.experimental.pallas import tpu_sc as plsc

N = 10000
E = 320000
H = 128
ROUNDS = 5

NC, NS, L = 2, 16, 16
NW = NC * NS
CH = 32
SB = 32
NSC = 10
NCHUNK = NSC * SB
EPW = NCHUNK * CH
EPAD = NW * EPW
RPT = N // NS

_MESH = plsc.VectorSubcoreMesh(core_axis_name="c", subcore_axis_name="s")


def _hsum(v):
    return plsc.cumsum(v)[15]


def _hmax(v):
    return plsc.cummax(v)[15]


def _minmax(a, b, i, mm_ref):
    new = jnp.concatenate(
        [jnp.max(a, axis=0, keepdims=True), jnp.min(a, axis=0, keepdims=True),
         jnp.max(b, axis=0, keepdims=True), jnp.min(b, axis=0, keepdims=True)],
        axis=0)
    cur = mm_ref[...]
    comb = jnp.concatenate(
        [jnp.maximum(cur[0:1], new[0:1]), jnp.minimum(cur[1:2], new[1:2]),
         jnp.maximum(cur[2:3], new[2:3]), jnp.minimum(cur[3:4], new[3:4])],
        axis=0)
    mm_ref[...] = jnp.where(i == 0, new, comb)


def _pre_body(x_ref, wpt, bp, wlt, wrt, ba1, h_ref, a_ref, b_ref, mm_ref):
    i = pl.program_id(0)
    h = jnp.maximum(jnp.dot(x_ref[...], wpt[...],
                            preferred_element_type=jnp.float32) + bp[...], 0.0)
    h_ref[...] = h
    a = jnp.dot(h, wlt[...], preferred_element_type=jnp.float32)
    b = jnp.dot(h, wrt[...], preferred_element_type=jnp.float32) + ba1[...]
    a_ref[...] = a
    b_ref[...] = b
    _minmax(a, b, i, mm_ref)


def _gru(h, num_ref, den_ref, wmax_ref, wiht, whht, bih, bhh):
    c = jnp.max(wmax_ref[...])
    num = num_ref[0] + num_ref[1]
    den = jnp.sum(den_ref[0] + den_ref[1], axis=-1, keepdims=True)
    agg = num / (den + 1e-8 * c)
    gi = jnp.dot(agg, wiht[...], preferred_element_type=jnp.float32) + bih[...]
    gh = jnp.dot(h, whht[...], preferred_element_type=jnp.float32) + bhh[...]
    r = jax.nn.sigmoid(gi[:, 0:H] + gh[:, 0:H])
    z = jax.nn.sigmoid(gi[:, H:2 * H] + gh[:, H:2 * H])
    n = jnp.tanh(gi[:, 2 * H:] + r * gh[:, 2 * H:])
    return (1.0 - z) * n + z * h


def _step_body(h_ref, num_ref, den_ref, wmax_ref, wiht, whht, bih, bhh,
               wlt, wrt, ba1, h_out, a_out, b_out, mm_ref):
    i = pl.program_id(0)
    hn = _gru(h_ref[...], num_ref, den_ref, wmax_ref, wiht, whht, bih, bhh)
    h_out[...] = hn
    a = jnp.dot(hn, wlt[...], preferred_element_type=jnp.float32)
    b = jnp.dot(hn, wrt[...], preferred_element_type=jnp.float32) + ba1[...]
    a_out[...] = a
    b_out[...] = b
    _minmax(a, b, i, mm_ref)


def _final_body(h_ref, num_ref, den_ref, wmax_ref, wiht, whht, bih, bhh,
                wot, bo, out_ref):
    hn = _gru(h_ref[...], num_ref, den_ref, wmax_ref, wiht, whht, bih, bhh)
    out_ref[...] = jnp.dot(hn, wot[...],
                           preferred_element_type=jnp.float32) + bo[...]


_BLK = 1000
_GRID = N // _BLK


def _row_spec():
    return pl.BlockSpec((_BLK, H), lambda i: (i, 0))


def _whole(shape):
    return pl.BlockSpec(shape, lambda i: tuple(0 for _ in shape))


def _tc_pre(x, wpt, bp, wlt, wrt, ba1):
    return pl.pallas_call(
        _pre_body,
        grid=(_GRID,),
        in_specs=[_row_spec(), _whole((H, H)), _whole((1, H)),
                  _whole((H, H)), _whole((H, H)), _whole((1, H))],
        out_specs=[_row_spec(), _row_spec(), _row_spec(), _whole((4, H))],
        out_shape=[jax.ShapeDtypeStruct((N, H), jnp.float32)] * 3 +
                  [jax.ShapeDtypeStruct((4, H), jnp.float32)],
    )(x, wpt, bp, wlt, wrt, ba1)


def _agg_specs():
    return [pl.BlockSpec((2, _BLK, H), lambda i: (0, i, 0)),
            pl.BlockSpec((2, _BLK, L), lambda i: (0, i, 0)),
            _whole((NW, L))]


def _tc_step(h, num, den, wmax, wiht, whht, bih, bhh, wlt, wrt, ba1):
    return pl.pallas_call(
        _step_body,
        grid=(_GRID,),
        in_specs=[_row_spec()] + _agg_specs() +
                 [_whole((H, 3 * H)), _whole((H, 3 * H)), _whole((1, 3 * H)),
                  _whole((1, 3 * H)), _whole((H, H)), _whole((H, H)),
                  _whole((1, H))],
        out_specs=[_row_spec(), _row_spec(), _row_spec(), _whole((4, H))],
        out_shape=[jax.ShapeDtypeStruct((N, H), jnp.float32)] * 3 +
                  [jax.ShapeDtypeStruct((4, H), jnp.float32)],
    )(h, num, den, wmax, wiht, whht, bih, bhh, wlt, wrt, ba1)


def _tc_final(h, num, den, wmax, wiht, whht, bih, bhh, wot, bo):
    return pl.pallas_call(
        _final_body,
        grid=(_GRID,),
        in_specs=[_row_spec()] + _agg_specs() +
                 [_whole((H, 3 * H)), _whole((H, 3 * H)), _whole((1, 3 * H)),
                  _whole((1, 3 * H)), _whole((H, H)), _whole((1, H))],
        out_specs=[_row_spec()],
        out_shape=[jax.ShapeDtypeStruct((N, H), jnp.float32)],
    )(h, num, den, wmax, wiht, whht, bih, bhh, wot, bo)[0]


@functools.partial(
    pl.kernel,
    out_type=[jax.ShapeDtypeStruct((NC, N, H), jnp.float32),
              jax.ShapeDtypeStruct((NC, N, L), jnp.float32),
              jax.ShapeDtypeStruct((NW, L), jnp.float32)],
    mesh=_MESH,
    compiler_params=pltpu.CompilerParams(needs_layout_passes=False,
                                         use_tc_tiling_on_sc=False),
    scratch_types=[
        pltpu.VMEM((SB, CH), jnp.int32),
        pltpu.VMEM((SB, CH), jnp.int32),
        pltpu.VMEM((CH, H), jnp.float32),
        pltpu.VMEM((CH, H), jnp.float32),
        pltpu.VMEM((CH, H), jnp.float32),
        pltpu.VMEM((CH, H), jnp.float32),
        pltpu.VMEM((CH, H), jnp.float32),
        pltpu.VMEM((CH, H), jnp.float32),
        pltpu.VMEM((CH, H), jnp.float32),
        pltpu.VMEM((CH, H), jnp.float32),
        pltpu.VMEM((CH, L), jnp.float32),
        pltpu.VMEM((CH, L), jnp.float32),
        pltpu.VMEM((H,), jnp.float32),
        pltpu.VMEM((4, H), jnp.float32),
        pltpu.VMEM_SHARED((N, H), jnp.float32),
        pltpu.VMEM_SHARED((N, L), jnp.float32),
        pltpu.SemaphoreType.DMA,
        pltpu.SemaphoreType.DMA,
        pltpu.SemaphoreType.DMA,
        pltpu.SemaphoreType.DMA,
        pltpu.SemaphoreType.DMA,
        pltpu.SemaphoreType.DMA,
        pltpu.SemaphoreType.DMA,
        pltpu.SemaphoreType.DMA,
        pltpu.SemaphoreType.DMA,
        pltpu.SemaphoreType.DMA,
        pltpu.SemaphoreType.DMA,
        pltpu.SemaphoreType.DMA,
        pltpu.SemaphoreType.DMA,
        pltpu.SemaphoreType.DMA,
    ],
)
def _sc_edge(a_hbm, b_hbm, h_hbm, wa2_hbm, mm_hbm, row_hbm, col_hbm,
             zn_hbm, zd_hbm, num_hbm, den_hbm, wmax_hbm,
             row_v, col_v,
             ga0, gb0, gh0, gv0, ga1, gb1, gh1, gv1, db0, db1,
             wa2_v, mm_v, acc_num, acc_den,
             sa0, sb0, sh0, sv0, sa1, sb1, sh1, sv1,
             sn0, sd0, sn1, sd1, si0, si1):
    c = lax.axis_index("c")
    sid = lax.axis_index("s")
    wid = sid * NC + c
    pltpu.sync_copy(wa2_hbm, wa2_v)
    pltpu.sync_copy(mm_hbm, mm_v)
    wvecs = [wa2_v[pl.ds(16 * k, 16)] for k in range(8)]
    lanes = lax.iota(jnp.int32, 16)
    masks = [lanes == l for l in range(16)]
    mask0 = masks[0]
    zero16 = jnp.zeros((16,), jnp.float32)

    ub = jnp.zeros((16,), jnp.float32)
    for k in range(8):
        wk = wvecs[k]
        hi_ab = jnp.maximum(mm_v[0, pl.ds(16 * k, 16)]
                            + mm_v[2, pl.ds(16 * k, 16)], 0.0)
        lo_ab = jnp.maximum(mm_v[1, pl.ds(16 * k, 16)]
                            + mm_v[3, pl.ds(16 * k, 16)], 0.0)
        ub = ub + jnp.where(wk >= 0.0, wk * hi_ab, wk * lo_ab)
    mhat = _hsum(ub)

    tbase = sid * RPT
    pltpu.sync_copy(zn_hbm, acc_num.at[pl.ds(tbase, RPT)])
    pltpu.sync_copy(zd_hbm, acc_den.at[pl.ds(tbase, RPT)])
    plsc.subcore_barrier()

    slots = ((ga0, gb0, gh0, gv0, db0, sa0, sb0, sh0, sv0, sn0, sd0),
             (ga1, gb1, gh1, gv1, db1, sa1, sb1, sh1, sv1, sn1, sd1))

    def issue_gather(jj, slot):
        ga, gb, gh, gv, _, sa, sb, sh, sv, _, _ = slots[slot]
        pltpu.async_copy(a_hbm.at[row_v.at[jj]], ga, sa)
        pltpu.async_copy(b_hbm.at[col_v.at[jj]], gb, sb)
        pltpu.async_copy(h_hbm.at[row_v.at[jj]], gh, sh)
        pltpu.async_copy(h_hbm.at[col_v.at[jj]], gv, sv)

    def wait_gather(slot):
        ga, gb, gh, gv, _, sa, sb, sh, sv, _, _ = slots[slot]
        pltpu.make_async_copy(a_hbm.at[row_v.at[0]], ga, sa).wait()
        pltpu.make_async_copy(b_hbm.at[col_v.at[0]], gb, sb).wait()
        pltpu.make_async_copy(h_hbm.at[row_v.at[0]], gh, sh).wait()
        pltpu.make_async_copy(h_hbm.at[col_v.at[0]], gv, sv).wait()

    def issue_scatter(jj, slot):
        _, _, _, gv, db, _, _, _, _, sn, sd = slots[slot]
        pltpu.async_copy(gv, acc_num.at[row_v.at[jj]], sn, add=True)
        pltpu.async_copy(db, acc_den.at[row_v.at[jj]], sd, add=True)

    def wait_scatter(slot):
        _, _, _, gv, db, _, _, _, _, sn, sd = slots[slot]
        pltpu.make_async_copy(gv, acc_num.at[row_v.at[0]], sn).wait()
        pltpu.make_async_copy(db, acc_den.at[row_v.at[0]], sd).wait()

    def compute(j, slot, wmax):
        ga, gb, gh, gv, db, _, _, _, _, _, _ = slots[slot]

        def group(g, wmax):
            svec = jnp.zeros((16,), jnp.float32)
            for l in range(16):
                e = g * 16 + l
                acc = jnp.zeros((16,), jnp.float32)
                for k in range(8):
                    va = ga[e, pl.ds(16 * k, 16)]
                    vb = gb[e, pl.ds(16 * k, 16)]
                    acc = acc + jnp.maximum(va + vb, 0.0) * wvecs[k]
                svec = jnp.where(masks[l], _hsum(acc), svec)
            base = wid * EPW + j * CH + g * 16
            wv = jnp.exp(svec - mhat)
            wv = jnp.where(lanes + base < E, wv, 0.0)
            for l in range(16):
                e = g * 16 + l
                ws = wv[l]
                db[e, pl.ds(0, 16)] = jnp.where(mask0, ws, zero16)
                for k in range(8):
                    d = jnp.abs(gh[e, pl.ds(16 * k, 16)]
                                - gv[e, pl.ds(16 * k, 16)])
                    gv[e, pl.ds(16 * k, 16)] = d * ws
            return jnp.maximum(wmax, _hmax(wv))

        return lax.fori_loop(0, CH // 16, group, wmax)

    def superchunk(sc, wmax):
        ci = pltpu.async_copy(row_hbm.at[wid, pl.ds(sc * SB, SB)], row_v, si0)
        cj = pltpu.async_copy(col_hbm.at[wid, pl.ds(sc * SB, SB)], col_v, si1)
        ci.wait()
        cj.wait()
        issue_gather(0, 0)
        issue_gather(1, 1)

        def pair(p, wmax):
            jj = 2 * p
            j = sc * SB + jj
            wait_gather(0)
            wmax = compute(j, 0, wmax)
            issue_scatter(jj, 0)
            wait_gather(1)
            wmax = compute(j + 1, 1, wmax)
            issue_scatter(jj + 1, 1)
            wait_scatter(0)

            @pl.when(jj + 2 < SB)
            def _():
                issue_gather(jj + 2, 0)

            wait_scatter(1)

            @pl.when(jj + 3 < SB)
            def _():
                issue_gather(jj + 3, 1)

            return wmax

        return lax.fori_loop(0, SB // 2, pair, wmax)

    wmax = lax.fori_loop(0, NSC, superchunk, jnp.float32(0.0))
    plsc.subcore_barrier()
    pltpu.sync_copy(acc_num.at[pl.ds(tbase, RPT)],
                    num_hbm.at[c, pl.ds(tbase, RPT)])
    pltpu.sync_copy(acc_den.at[pl.ds(tbase, RPT)],
                    den_hbm.at[c, pl.ds(tbase, RPT)])
    db0[0, pl.ds(0, 16)] = jnp.broadcast_to(wmax, (16,))
    pltpu.sync_copy(db0.at[0, pl.ds(0, 16)], wmax_hbm.at[wid])


def kernel(x, edge_index, Wp, bp, Wa1, ba1, Wa2, ba2, Wih, Whh, bih, bhh,
           Wo, bo):
    del ba2
    row = edge_index[0].astype(jnp.int32)
    col = edge_index[1].astype(jnp.int32)
    pad = jnp.zeros((EPAD - E,), jnp.int32)
    rowp = jnp.concatenate([row, pad]).reshape(NW, NCHUNK, CH)
    colp = jnp.concatenate([col, pad]).reshape(NW, NCHUNK, CH)
    zn = jnp.zeros((RPT, H), jnp.float32)
    zd = jnp.zeros((RPT, L), jnp.float32)

    wpt = Wp.T
    wlt = Wa1[:, :H].T
    wrt = Wa1[:, H:].T
    wa2v = Wa2.reshape(H)
    wiht = Wih.T
    whht = Whh.T
    wot = Wo.T
    bp2 = bp.reshape(1, H)
    ba12 = ba1.reshape(1, H)
    bih2 = bih.reshape(1, 3 * H)
    bhh2 = bhh.reshape(1, 3 * H)
    bo2 = bo.reshape(1, H)

    h, a, b, mm = _tc_pre(x, wpt, bp2, wlt, wrt, ba12)
    for r in range(ROUNDS):
        num, den, wmax = _sc_edge(a, b, h, wa2v, mm, rowp, colp, zn, zd)
        if r < ROUNDS - 1:
            h, a, b, mm = _tc_step(h, num, den, wmax, wiht, whht, bih2, bhh2,
                                   wlt, wrt, ba12)
        else:
            out = _tc_final(h, num, den, wmax, wiht, whht, bih2, bhh2,
                            wot, bo2)
    return out


# speedup vs baseline: 2.9245x; 1.0496x over previous
import functools

import jax
import jax.numpy as jnp
from jax import lax
from jax.experimental import pallas as pl
from jax.experimental.pallas import tpu as pltpu
from jax.experimental.pallas import tpu_sc as plsc

N = 10000
E = 320000
H = 128
ROUNDS = 5

NC, NS, L = 2, 16, 16
NW = NC * NS
CH = 32
SB = 32
NSC = 10
NCHUNK = NSC * SB
EPW = NCHUNK * CH
EPAD = NW * EPW
RPT = N // NS

_MESH = plsc.VectorSubcoreMesh(core_axis_name="c", subcore_axis_name="s")


def _hsum(v):
    return plsc.cumsum(v)[15]


def _hmax(v):
    return plsc.cummax(v)[15]


def _minmax(a, b, i, mm_ref):
    new = jnp.concatenate(
        [jnp.max(a, axis=0, keepdims=True), jnp.min(a, axis=0, keepdims=True),
         jnp.max(b, axis=0, keepdims=True), jnp.min(b, axis=0, keepdims=True)],
        axis=0)
    cur = mm_ref[...]
    comb = jnp.concatenate(
        [jnp.maximum(cur[0:1], new[0:1]), jnp.minimum(cur[1:2], new[1:2]),
         jnp.maximum(cur[2:3], new[2:3]), jnp.minimum(cur[3:4], new[3:4])],
        axis=0)
    mm_ref[...] = jnp.where(i == 0, new, comb)


def _pre_body(x_ref, wpt, bp, wlt, wrt, ba1, h_ref, a_ref, b_ref, mm_ref):
    i = pl.program_id(0)
    h = jnp.maximum(jnp.dot(x_ref[...], wpt[...],
                            preferred_element_type=jnp.float32) + bp[...], 0.0)
    h_ref[...] = h
    a = jnp.dot(h, wlt[...], preferred_element_type=jnp.float32)
    b = jnp.dot(h, wrt[...], preferred_element_type=jnp.float32) + ba1[...]
    a_ref[...] = a
    b_ref[...] = b
    _minmax(a, b, i, mm_ref)


def _gru(h, num_ref, den_ref, wmax_ref, wiht, whht, bih, bhh):
    c = jnp.max(wmax_ref[...])
    num = num_ref[0] + num_ref[1]
    den = jnp.sum(den_ref[0] + den_ref[1], axis=-1, keepdims=True)
    agg = num / (den + 1e-8 * c)
    gi = jnp.dot(agg, wiht[...], preferred_element_type=jnp.float32) + bih[...]
    gh = jnp.dot(h, whht[...], preferred_element_type=jnp.float32) + bhh[...]
    r = jax.nn.sigmoid(gi[:, 0:H] + gh[:, 0:H])
    z = jax.nn.sigmoid(gi[:, H:2 * H] + gh[:, H:2 * H])
    n = jnp.tanh(gi[:, 2 * H:] + r * gh[:, 2 * H:])
    return (1.0 - z) * n + z * h


def _step_body(h_ref, num_ref, den_ref, wmax_ref, wiht, whht, bih, bhh,
               wlt, wrt, ba1, h_out, a_out, b_out, mm_ref):
    i = pl.program_id(0)
    hn = _gru(h_ref[...], num_ref, den_ref, wmax_ref, wiht, whht, bih, bhh)
    h_out[...] = hn
    a = jnp.dot(hn, wlt[...], preferred_element_type=jnp.float32)
    b = jnp.dot(hn, wrt[...], preferred_element_type=jnp.float32) + ba1[...]
    a_out[...] = a
    b_out[...] = b
    _minmax(a, b, i, mm_ref)


def _final_body(h_ref, num_ref, den_ref, wmax_ref, wiht, whht, bih, bhh,
                wot, bo, out_ref):
    hn = _gru(h_ref[...], num_ref, den_ref, wmax_ref, wiht, whht, bih, bhh)
    out_ref[...] = jnp.dot(hn, wot[...],
                           preferred_element_type=jnp.float32) + bo[...]


_BLK = 1000
_GRID = N // _BLK


def _row_spec():
    return pl.BlockSpec((_BLK, H), lambda i: (i, 0))


def _whole(shape):
    return pl.BlockSpec(shape, lambda i: tuple(0 for _ in shape))


def _tc_pre(x, wpt, bp, wlt, wrt, ba1):
    return pl.pallas_call(
        _pre_body,
        grid=(_GRID,),
        in_specs=[_row_spec(), _whole((H, H)), _whole((1, H)),
                  _whole((H, H)), _whole((H, H)), _whole((1, H))],
        out_specs=[_row_spec(), _row_spec(), _row_spec(), _whole((4, H))],
        out_shape=[jax.ShapeDtypeStruct((N, H), jnp.float32)] * 3 +
                  [jax.ShapeDtypeStruct((4, H), jnp.float32)],
    )(x, wpt, bp, wlt, wrt, ba1)


def _agg_specs():
    return [pl.BlockSpec((2, _BLK, H), lambda i: (0, i, 0)),
            pl.BlockSpec((2, _BLK, L), lambda i: (0, i, 0)),
            _whole((NW, L))]


def _tc_step(h, num, den, wmax, wiht, whht, bih, bhh, wlt, wrt, ba1):
    return pl.pallas_call(
        _step_body,
        grid=(_GRID,),
        in_specs=[_row_spec()] + _agg_specs() +
                 [_whole((H, 3 * H)), _whole((H, 3 * H)), _whole((1, 3 * H)),
                  _whole((1, 3 * H)), _whole((H, H)), _whole((H, H)),
                  _whole((1, H))],
        out_specs=[_row_spec(), _row_spec(), _row_spec(), _whole((4, H))],
        out_shape=[jax.ShapeDtypeStruct((N, H), jnp.float32)] * 3 +
                  [jax.ShapeDtypeStruct((4, H), jnp.float32)],
    )(h, num, den, wmax, wiht, whht, bih, bhh, wlt, wrt, ba1)


def _tc_final(h, num, den, wmax, wiht, whht, bih, bhh, wot, bo):
    return pl.pallas_call(
        _final_body,
        grid=(_GRID,),
        in_specs=[_row_spec()] + _agg_specs() +
                 [_whole((H, 3 * H)), _whole((H, 3 * H)), _whole((1, 3 * H)),
                  _whole((1, 3 * H)), _whole((H, H)), _whole((1, H))],
        out_specs=[_row_spec()],
        out_shape=[jax.ShapeDtypeStruct((N, H), jnp.float32)],
    )(h, num, den, wmax, wiht, whht, bih, bhh, wot, bo)[0]


@functools.partial(
    pl.kernel,
    out_type=[jax.ShapeDtypeStruct((NC, N, H), jnp.float32),
              jax.ShapeDtypeStruct((NC, N, L), jnp.float32),
              jax.ShapeDtypeStruct((NW, L), jnp.float32)],
    mesh=_MESH,
    compiler_params=pltpu.CompilerParams(needs_layout_passes=False,
                                         use_tc_tiling_on_sc=False),
    scratch_types=[
        pltpu.VMEM((SB, CH), jnp.int32),
        pltpu.VMEM((SB, CH), jnp.int32),
        pltpu.VMEM((CH, H), jnp.float32),
        pltpu.VMEM((CH, H), jnp.float32),
        pltpu.VMEM((CH, H), jnp.float32),
        pltpu.VMEM((CH, H), jnp.float32),
        pltpu.VMEM((CH, H), jnp.float32),
        pltpu.VMEM((CH, H), jnp.float32),
        pltpu.VMEM((CH, H), jnp.float32),
        pltpu.VMEM((CH, H), jnp.float32),
        pltpu.VMEM((CH, L), jnp.float32),
        pltpu.VMEM((CH, L), jnp.float32),
        pltpu.VMEM((H,), jnp.float32),
        pltpu.VMEM((4, H), jnp.float32),
        pltpu.VMEM_SHARED((N, H), jnp.float32),
        pltpu.VMEM_SHARED((N, L), jnp.float32),
        pltpu.SemaphoreType.DMA,
        pltpu.SemaphoreType.DMA,
        pltpu.SemaphoreType.DMA,
        pltpu.SemaphoreType.DMA,
        pltpu.SemaphoreType.DMA,
        pltpu.SemaphoreType.DMA,
        pltpu.SemaphoreType.DMA,
        pltpu.SemaphoreType.DMA,
        pltpu.SemaphoreType.DMA,
        pltpu.SemaphoreType.DMA,
        pltpu.SemaphoreType.DMA,
        pltpu.SemaphoreType.DMA,
        pltpu.SemaphoreType.DMA,
        pltpu.SemaphoreType.DMA,
    ],
)
def _sc_edge(a_hbm, b_hbm, h_hbm, wa2_hbm, mm_hbm, row_hbm, col_hbm,
             zn_hbm, zd_hbm, num_hbm, den_hbm, wmax_hbm,
             row_v, col_v,
             ga0, gb0, gh0, gv0, ga1, gb1, gh1, gv1, db0, db1,
             wa2_v, mm_v, acc_num, acc_den,
             sa0, sb0, sh0, sv0, sa1, sb1, sh1, sv1,
             sn0, sd0, sn1, sd1, si0, si1):
    c = lax.axis_index("c")
    sid = lax.axis_index("s")
    wid = sid * NC + c
    pltpu.sync_copy(wa2_hbm, wa2_v)
    pltpu.sync_copy(mm_hbm, mm_v)
    wvecs = [wa2_v[pl.ds(16 * k, 16)] for k in range(8)]
    lanes = lax.iota(jnp.int32, 16)
    masks = [lanes == l for l in range(16)]
    mask0 = masks[0]
    zero16 = jnp.zeros((16,), jnp.float32)

    ub = jnp.zeros((16,), jnp.float32)
    for k in range(8):
        wk = wvecs[k]
        hi_ab = jnp.maximum(mm_v[0, pl.ds(16 * k, 16)]
                            + mm_v[2, pl.ds(16 * k, 16)], 0.0)
        lo_ab = jnp.maximum(mm_v[1, pl.ds(16 * k, 16)]
                            + mm_v[3, pl.ds(16 * k, 16)], 0.0)
        ub = ub + jnp.where(wk >= 0.0, wk * hi_ab, wk * lo_ab)
    mhat = _hsum(ub)

    tbase = sid * RPT
    pltpu.sync_copy(zn_hbm, acc_num.at[pl.ds(tbase, RPT)])
    pltpu.sync_copy(zd_hbm, acc_den.at[pl.ds(tbase, RPT)])
    plsc.subcore_barrier()

    slots = ((ga0, gb0, gh0, gv0, db0, sa0, sb0, sh0, sv0, sn0, sd0),
             (ga1, gb1, gh1, gv1, db1, sa1, sb1, sh1, sv1, sn1, sd1))

    def issue_gather_abh(jj, slot):
        ga, gb, gh, _, _, sa, sb, sh, _, _, _ = slots[slot]
        pltpu.async_copy(a_hbm.at[row_v.at[jj]], ga, sa)
        pltpu.async_copy(b_hbm.at[col_v.at[jj]], gb, sb)
        pltpu.async_copy(h_hbm.at[row_v.at[jj]], gh, sh)

    def issue_gather_v(jj, slot):
        _, _, _, gv, _, _, _, _, sv, _, _ = slots[slot]
        pltpu.async_copy(h_hbm.at[col_v.at[jj]], gv, sv)

    def issue_gather(jj, slot):
        issue_gather_abh(jj, slot)
        issue_gather_v(jj, slot)

    def wait_gather(slot):
        ga, gb, gh, gv, _, sa, sb, sh, sv, _, _ = slots[slot]
        pltpu.make_async_copy(a_hbm.at[row_v.at[0]], ga, sa).wait()
        pltpu.make_async_copy(b_hbm.at[col_v.at[0]], gb, sb).wait()
        pltpu.make_async_copy(h_hbm.at[row_v.at[0]], gh, sh).wait()
        pltpu.make_async_copy(h_hbm.at[col_v.at[0]], gv, sv).wait()

    def issue_scatter(jj, slot):
        _, _, _, gv, db, _, _, _, _, sn, sd = slots[slot]
        pltpu.async_copy(gv, acc_num.at[row_v.at[jj]], sn, add=True)
        pltpu.async_copy(db, acc_den.at[row_v.at[jj]], sd, add=True)

    def wait_scatter(slot):
        _, _, _, gv, db, _, _, _, _, sn, sd = slots[slot]
        pltpu.make_async_copy(gv, acc_num.at[row_v.at[0]], sn).wait()
        pltpu.make_async_copy(db, acc_den.at[row_v.at[0]], sd).wait()

    def compute(j, slot, wmax):
        ga, gb, gh, gv, db, _, _, _, _, _, _ = slots[slot]

        def group(g, wmax):
            svec = jnp.zeros((16,), jnp.float32)
            for l in range(16):
                e = g * 16 + l
                acc = jnp.zeros((16,), jnp.float32)
                for k in range(8):
                    va = ga[e, pl.ds(16 * k, 16)]
                    vb = gb[e, pl.ds(16 * k, 16)]
                    acc = acc + jnp.maximum(va + vb, 0.0) * wvecs[k]
                svec = jnp.where(masks[l], _hsum(acc), svec)
            base = wid * EPW + j * CH + g * 16
            wv = jnp.exp(svec - mhat)
            wv = jnp.where(lanes + base < E, wv, 0.0)
            for l in range(16):
                e = g * 16 + l
                ws = wv[l]
                db[e, pl.ds(0, 16)] = jnp.where(mask0, ws, zero16)
                for k in range(8):
                    d = jnp.abs(gh[e, pl.ds(16 * k, 16)]
                                - gv[e, pl.ds(16 * k, 16)])
                    gv[e, pl.ds(16 * k, 16)] = d * ws
            return jnp.maximum(wmax, _hmax(wv))

        return lax.fori_loop(0, CH // 16, group, wmax)

    def superchunk(sc, wmax):
        ci = pltpu.async_copy(row_hbm.at[wid, pl.ds(sc * SB, SB)], row_v, si0)
        cj = pltpu.async_copy(col_hbm.at[wid, pl.ds(sc * SB, SB)], col_v, si1)
        ci.wait()
        cj.wait()
        issue_gather(0, 0)
        issue_gather(1, 1)

        def pair(p, wmax):
            jj = 2 * p
            j = sc * SB + jj
            wait_gather(0)
            wmax = compute(j, 0, wmax)
            issue_scatter(jj, 0)

            @pl.when(jj + 2 < SB)
            def _():
                issue_gather_abh(jj + 2, 0)

            wait_gather(1)
            wmax = compute(j + 1, 1, wmax)
            issue_scatter(jj + 1, 1)

            @pl.when(jj + 3 < SB)
            def _():
                issue_gather_abh(jj + 3, 1)

            wait_scatter(0)

            @pl.when(jj + 2 < SB)
            def _():
                issue_gather_v(jj + 2, 0)

            wait_scatter(1)

            @pl.when(jj + 3 < SB)
            def _():
                issue_gather_v(jj + 3, 1)

            return wmax

        return lax.fori_loop(0, SB // 2, pair, wmax)

    wmax = lax.fori_loop(0, NSC, superchunk, jnp.float32(0.0))
    plsc.subcore_barrier()
    pltpu.sync_copy(acc_num.at[pl.ds(tbase, RPT)],
                    num_hbm.at[c, pl.ds(tbase, RPT)])
    pltpu.sync_copy(acc_den.at[pl.ds(tbase, RPT)],
                    den_hbm.at[c, pl.ds(tbase, RPT)])
    db0[0, pl.ds(0, 16)] = jnp.broadcast_to(wmax, (16,))
    pltpu.sync_copy(db0.at[0, pl.ds(0, 16)], wmax_hbm.at[wid])


def kernel(x, edge_index, Wp, bp, Wa1, ba1, Wa2, ba2, Wih, Whh, bih, bhh,
           Wo, bo):
    del ba2
    row = edge_index[0].astype(jnp.int32)
    col = edge_index[1].astype(jnp.int32)
    pad = jnp.zeros((EPAD - E,), jnp.int32)
    rowp = jnp.concatenate([row, pad]).reshape(NW, NCHUNK, CH)
    colp = jnp.concatenate([col, pad]).reshape(NW, NCHUNK, CH)
    zn = jnp.zeros((RPT, H), jnp.float32)
    zd = jnp.zeros((RPT, L), jnp.float32)

    wpt = Wp.T
    wlt = Wa1[:, :H].T
    wrt = Wa1[:, H:].T
    wa2v = Wa2.reshape(H)
    wiht = Wih.T
    whht = Whh.T
    wot = Wo.T
    bp2 = bp.reshape(1, H)
    ba12 = ba1.reshape(1, H)
    bih2 = bih.reshape(1, 3 * H)
    bhh2 = bhh.reshape(1, 3 * H)
    bo2 = bo.reshape(1, H)

    h, a, b, mm = _tc_pre(x, wpt, bp2, wlt, wrt, ba12)
    for r in range(ROUNDS):
        num, den, wmax = _sc_edge(a, b, h, wa2v, mm, rowp, colp, zn, zd)
        if r < ROUNDS - 1:
            h, a, b, mm = _tc_step(h, num, den, wmax, wiht, whht, bih2, bhh2,
                                   wlt, wrt, ba12)
        else:
            out = _tc_final(h, num, den, wmax, wiht, whht, bih2, bhh2,
                            wot, bo2)
    return out


# speedup vs baseline: 3.4817x; 1.1906x over previous
import functools

import jax
import jax.numpy as jnp
from jax import lax
from jax.experimental import pallas as pl
from jax.experimental.pallas import tpu as pltpu
from jax.experimental.pallas import tpu_sc as plsc

N = 10000
E = 320000
H = 128
ROUNDS = 5

NC, NS, L = 2, 16, 16
NW = NC * NS
CH = 32
SB = 32
NSC = 10
NCHUNK = NSC * SB
EPW = NCHUNK * CH
EPAD = NW * EPW
RPT = N // NS

_MESH = plsc.VectorSubcoreMesh(core_axis_name="c", subcore_axis_name="s")


def _hsum(v):
    return plsc.cumsum(v)[15]


def _hmax(v):
    return plsc.cummax(v)[15]


def _minmax(a, b, i, mm_ref):
    new = jnp.concatenate(
        [jnp.max(a, axis=0, keepdims=True), jnp.min(a, axis=0, keepdims=True),
         jnp.max(b, axis=0, keepdims=True), jnp.min(b, axis=0, keepdims=True)],
        axis=0)
    cur = mm_ref[...]
    comb = jnp.concatenate(
        [jnp.maximum(cur[0:1], new[0:1]), jnp.minimum(cur[1:2], new[1:2]),
         jnp.maximum(cur[2:3], new[2:3]), jnp.minimum(cur[3:4], new[3:4])],
        axis=0)
    mm_ref[...] = jnp.where(i == 0, new, comb)


def _pre_body(x_ref, wpt, bp, wlt, wrt, ba1, h_ref, t_ref, u_ref, mm_ref):
    i = pl.program_id(0)
    h = jnp.maximum(jnp.dot(x_ref[...], wpt[...],
                            preferred_element_type=jnp.float32) + bp[...], 0.0)
    h_ref[...] = h
    a = jnp.dot(h, wlt[...], preferred_element_type=jnp.float32)
    b = jnp.dot(h, wrt[...], preferred_element_type=jnp.float32) + ba1[...]
    t_ref[...] = jnp.concatenate([a, h], axis=1)
    u_ref[...] = b
    _minmax(a, b, i, mm_ref)


def _gru(h, num_ref, den_ref, wmax_ref, wiht, whht, bih, bhh):
    c = jnp.max(wmax_ref[...])
    num = num_ref[0] + num_ref[1]
    den = jnp.sum(den_ref[0] + den_ref[1], axis=-1, keepdims=True)
    agg = num / (den + 1e-8 * c)
    gi = jnp.dot(agg, wiht[...], preferred_element_type=jnp.float32) + bih[...]
    gh = jnp.dot(h, whht[...], preferred_element_type=jnp.float32) + bhh[...]
    r = jax.nn.sigmoid(gi[:, 0:H] + gh[:, 0:H])
    z = jax.nn.sigmoid(gi[:, H:2 * H] + gh[:, H:2 * H])
    n = jnp.tanh(gi[:, 2 * H:] + r * gh[:, 2 * H:])
    return (1.0 - z) * n + z * h


def _step_body(h_ref, num_ref, den_ref, wmax_ref, wiht, whht, bih, bhh,
               wlt, wrt, ba1, h_out, t_out, u_out, mm_ref):
    i = pl.program_id(0)
    hn = _gru(h_ref[...], num_ref, den_ref, wmax_ref, wiht, whht, bih, bhh)
    h_out[...] = hn
    a = jnp.dot(hn, wlt[...], preferred_element_type=jnp.float32)
    b = jnp.dot(hn, wrt[...], preferred_element_type=jnp.float32) + ba1[...]
    t_out[...] = jnp.concatenate([a, hn], axis=1)
    u_out[...] = b
    _minmax(a, b, i, mm_ref)


def _final_body(h_ref, num_ref, den_ref, wmax_ref, wiht, whht, bih, bhh,
                wot, bo, out_ref):
    hn = _gru(h_ref[...], num_ref, den_ref, wmax_ref, wiht, whht, bih, bhh)
    out_ref[...] = jnp.dot(hn, wot[...],
                           preferred_element_type=jnp.float32) + bo[...]


_BLK = 1000
_GRID = N // _BLK


def _row_spec():
    return pl.BlockSpec((_BLK, H), lambda i: (i, 0))


def _row_spec2():
    return pl.BlockSpec((_BLK, 2 * H), lambda i: (i, 0))


def _whole(shape):
    return pl.BlockSpec(shape, lambda i: tuple(0 for _ in shape))


def _tc_pre(x, wpt, bp, wlt, wrt, ba1):
    return pl.pallas_call(
        _pre_body,
        grid=(_GRID,),
        in_specs=[_row_spec(), _whole((H, H)), _whole((1, H)),
                  _whole((H, H)), _whole((H, H)), _whole((1, H))],
        out_specs=[_row_spec(), _row_spec2(), _row_spec(), _whole((4, H))],
        out_shape=[jax.ShapeDtypeStruct((N, H), jnp.float32),
                   jax.ShapeDtypeStruct((N, 2 * H), jnp.float32),
                   jax.ShapeDtypeStruct((N, H), jnp.float32),
                   jax.ShapeDtypeStruct((4, H), jnp.float32)],
    )(x, wpt, bp, wlt, wrt, ba1)


def _agg_specs():
    return [pl.BlockSpec((2, _BLK, H), lambda i: (0, i, 0)),
            pl.BlockSpec((2, _BLK, L), lambda i: (0, i, 0)),
            _whole((NW, L))]


def _tc_step(h, num, den, wmax, wiht, whht, bih, bhh, wlt, wrt, ba1):
    return pl.pallas_call(
        _step_body,
        grid=(_GRID,),
        in_specs=[_row_spec()] + _agg_specs() +
                 [_whole((H, 3 * H)), _whole((H, 3 * H)), _whole((1, 3 * H)),
                  _whole((1, 3 * H)), _whole((H, H)), _whole((H, H)),
                  _whole((1, H))],
        out_specs=[_row_spec(), _row_spec2(), _row_spec(), _whole((4, H))],
        out_shape=[jax.ShapeDtypeStruct((N, H), jnp.float32),
                   jax.ShapeDtypeStruct((N, 2 * H), jnp.float32),
                   jax.ShapeDtypeStruct((N, H), jnp.float32),
                   jax.ShapeDtypeStruct((4, H), jnp.float32)],
    )(h, num, den, wmax, wiht, whht, bih, bhh, wlt, wrt, ba1)


def _tc_final(h, num, den, wmax, wiht, whht, bih, bhh, wot, bo):
    return pl.pallas_call(
        _final_body,
        grid=(_GRID,),
        in_specs=[_row_spec()] + _agg_specs() +
                 [_whole((H, 3 * H)), _whole((H, 3 * H)), _whole((1, 3 * H)),
                  _whole((1, 3 * H)), _whole((H, H)), _whole((1, H))],
        out_specs=[_row_spec()],
        out_shape=[jax.ShapeDtypeStruct((N, H), jnp.float32)],
    )(h, num, den, wmax, wiht, whht, bih, bhh, wot, bo)[0]


@functools.partial(
    pl.kernel,
    out_type=[jax.ShapeDtypeStruct((NC, N, H), jnp.float32),
              jax.ShapeDtypeStruct((NC, N, L), jnp.float32),
              jax.ShapeDtypeStruct((NW, L), jnp.float32)],
    mesh=_MESH,
    compiler_params=pltpu.CompilerParams(needs_layout_passes=False,
                                         use_tc_tiling_on_sc=False),
    scratch_types=[
        pltpu.VMEM((SB, CH), jnp.int32),
        pltpu.VMEM((SB, CH), jnp.int32),
        pltpu.VMEM((CH, 2 * H), jnp.float32),
        pltpu.VMEM((CH, 2 * H), jnp.float32),
        pltpu.VMEM((CH, H), jnp.float32),
        pltpu.VMEM((CH, H), jnp.float32),
        pltpu.VMEM((CH, H), jnp.float32),
        pltpu.VMEM((CH, H), jnp.float32),
        pltpu.VMEM((CH, L), jnp.float32),
        pltpu.VMEM((CH, L), jnp.float32),
        pltpu.VMEM((H,), jnp.float32),
        pltpu.VMEM((4, H), jnp.float32),
        pltpu.VMEM_SHARED((N, H), jnp.float32),
        pltpu.VMEM_SHARED((N, L), jnp.float32),
        pltpu.SemaphoreType.DMA,
        pltpu.SemaphoreType.DMA,
        pltpu.SemaphoreType.DMA,
        pltpu.SemaphoreType.DMA,
        pltpu.SemaphoreType.DMA,
        pltpu.SemaphoreType.DMA,
        pltpu.SemaphoreType.DMA,
        pltpu.SemaphoreType.DMA,
        pltpu.SemaphoreType.DMA,
        pltpu.SemaphoreType.DMA,
        pltpu.SemaphoreType.DMA,
        pltpu.SemaphoreType.DMA,
    ],
)
def _sc_edge(t_hbm, b_hbm, h_hbm, wa2_hbm, mm_hbm, row_hbm, col_hbm,
             zn_hbm, zd_hbm, num_hbm, den_hbm, wmax_hbm,
             row_v, col_v, gt0, gt1, gb0, gb1, gv0, gv1, db0, db1,
             wa2_v, mm_v, acc_num, acc_den,
             st0, su0, st1, su1, sv0, sv1,
             sn0, sd0, sn1, sd1, si0, si1):
    c = lax.axis_index("c")
    sid = lax.axis_index("s")
    wid = sid * NC + c
    pltpu.sync_copy(wa2_hbm, wa2_v)
    pltpu.sync_copy(mm_hbm, mm_v)
    wvecs = [wa2_v[pl.ds(16 * k, 16)] for k in range(8)]
    lanes = lax.iota(jnp.int32, 16)
    masks = [lanes == l for l in range(16)]
    mask0 = masks[0]
    zero16 = jnp.zeros((16,), jnp.float32)

    ub = jnp.zeros((16,), jnp.float32)
    for k in range(8):
        wk = wvecs[k]
        hi_ab = jnp.maximum(mm_v[0, pl.ds(16 * k, 16)]
                            + mm_v[2, pl.ds(16 * k, 16)], 0.0)
        lo_ab = jnp.maximum(mm_v[1, pl.ds(16 * k, 16)]
                            + mm_v[3, pl.ds(16 * k, 16)], 0.0)
        ub = ub + jnp.where(wk >= 0.0, wk * hi_ab, wk * lo_ab)
    mhat = _hsum(ub)

    tbase = sid * RPT
    pltpu.sync_copy(zn_hbm, acc_num.at[pl.ds(tbase, RPT)])
    pltpu.sync_copy(zd_hbm, acc_den.at[pl.ds(tbase, RPT)])
    plsc.subcore_barrier()

    slots = ((gt0, gb0, gv0, db0, st0, su0, sv0, sn0, sd0),
             (gt1, gb1, gv1, db1, st1, su1, sv1, sn1, sd1))

    def issue_gather_t(jj, slot):
        gt, gb, _, _, st, su, _, _, _ = slots[slot]
        pltpu.async_copy(t_hbm.at[row_v.at[jj]], gt, st)
        pltpu.async_copy(b_hbm.at[col_v.at[jj]], gb, su)

    def issue_gather_u(jj, slot):
        _, _, gv, _, _, _, sv, _, _ = slots[slot]
        pltpu.async_copy(h_hbm.at[col_v.at[jj]], gv, sv)

    def wait_gather(slot):
        gt, gb, gv, _, st, su, sv, _, _ = slots[slot]
        pltpu.make_async_copy(t_hbm.at[row_v.at[0]], gt, st).wait()
        pltpu.make_async_copy(b_hbm.at[col_v.at[0]], gb, su).wait()
        pltpu.make_async_copy(h_hbm.at[col_v.at[0]], gv, sv).wait()

    def issue_scatter(jj, slot):
        _, _, gv, db, _, _, _, sn, sd = slots[slot]
        pltpu.async_copy(gv, acc_num.at[row_v.at[jj]], sn, add=True)
        pltpu.async_copy(db, acc_den.at[row_v.at[jj]], sd, add=True)

    def wait_scatter(slot):
        _, _, gv, db, _, _, _, sn, sd = slots[slot]
        pltpu.make_async_copy(gv, acc_num.at[row_v.at[0]], sn).wait()
        pltpu.make_async_copy(db, acc_den.at[row_v.at[0]], sd).wait()

    def compute(j, slot, wmax):
        gt, gb, gv, db, _, _, _, _, _ = slots[slot]

        def group(g, wmax):
            svec = jnp.zeros((16,), jnp.float32)
            for l in range(16):
                e = g * 16 + l
                acc = jnp.zeros((16,), jnp.float32)
                for k in range(8):
                    va = gt[e, pl.ds(16 * k, 16)]
                    vb = gb[e, pl.ds(16 * k, 16)]
                    acc = acc + jnp.maximum(va + vb, 0.0) * wvecs[k]
                svec = jnp.where(masks[l], _hsum(acc), svec)
            base = wid * EPW + j * CH + g * 16
            wv = jnp.exp(svec - mhat)
            wv = jnp.where(lanes + base < E, wv, 0.0)
            for l in range(16):
                e = g * 16 + l
                ws = wv[l]
                db[e, pl.ds(0, 16)] = jnp.where(mask0, ws, zero16)
                for k in range(8):
                    d = jnp.abs(gt[e, pl.ds(H + 16 * k, 16)]
                                - gv[e, pl.ds(16 * k, 16)])
                    gv[e, pl.ds(16 * k, 16)] = d * ws
            return jnp.maximum(wmax, _hmax(wv))

        return lax.fori_loop(0, CH // 16, group, wmax)

    def superchunk(sc, wmax):
        ci = pltpu.async_copy(row_hbm.at[wid, pl.ds(sc * SB, SB)], row_v, si0)
        cj = pltpu.async_copy(col_hbm.at[wid, pl.ds(sc * SB, SB)], col_v, si1)
        ci.wait()
        cj.wait()
        issue_gather_t(0, 0)
        issue_gather_u(0, 0)
        issue_gather_t(1, 1)
        issue_gather_u(1, 1)

        def pair(p, wmax):
            jj = 2 * p
            j = sc * SB + jj
            wait_gather(0)
            wmax = compute(j, 0, wmax)
            issue_scatter(jj, 0)

            @pl.when(jj + 2 < SB)
            def _():
                issue_gather_t(jj + 2, 0)

            wait_gather(1)
            wmax = compute(j + 1, 1, wmax)
            issue_scatter(jj + 1, 1)

            @pl.when(jj + 3 < SB)
            def _():
                issue_gather_t(jj + 3, 1)

            wait_scatter(0)

            @pl.when(jj + 2 < SB)
            def _():
                issue_gather_u(jj + 2, 0)

            wait_scatter(1)

            @pl.when(jj + 3 < SB)
            def _():
                issue_gather_u(jj + 3, 1)

            return wmax

        return lax.fori_loop(0, SB // 2, pair, wmax)

    wmax = lax.fori_loop(0, NSC, superchunk, jnp.float32(0.0))
    plsc.subcore_barrier()
    pltpu.sync_copy(acc_num.at[pl.ds(tbase, RPT)],
                    num_hbm.at[c, pl.ds(tbase, RPT)])
    pltpu.sync_copy(acc_den.at[pl.ds(tbase, RPT)],
                    den_hbm.at[c, pl.ds(tbase, RPT)])
    db0[0, pl.ds(0, 16)] = jnp.broadcast_to(wmax, (16,))
    pltpu.sync_copy(db0.at[0, pl.ds(0, 16)], wmax_hbm.at[wid])


def kernel(x, edge_index, Wp, bp, Wa1, ba1, Wa2, ba2, Wih, Whh, bih, bhh,
           Wo, bo):
    del ba2
    row = edge_index[0].astype(jnp.int32)
    col = edge_index[1].astype(jnp.int32)
    pad = jnp.zeros((EPAD - E,), jnp.int32)
    rowp = jnp.concatenate([row, pad]).reshape(NW, NCHUNK, CH)
    colp = jnp.concatenate([col, pad]).reshape(NW, NCHUNK, CH)
    zn = jnp.zeros((RPT, H), jnp.float32)
    zd = jnp.zeros((RPT, L), jnp.float32)

    wpt = Wp.T
    wlt = Wa1[:, :H].T
    wrt = Wa1[:, H:].T
    wa2v = Wa2.reshape(H)
    wiht = Wih.T
    whht = Whh.T
    wot = Wo.T
    bp2 = bp.reshape(1, H)
    ba12 = ba1.reshape(1, H)
    bih2 = bih.reshape(1, 3 * H)
    bhh2 = bhh.reshape(1, 3 * H)
    bo2 = bo.reshape(1, H)

    h, t, u, mm = _tc_pre(x, wpt, bp2, wlt, wrt, ba12)
    for r in range(ROUNDS):
        num, den, wmax = _sc_edge(t, u, h, wa2v, mm, rowp, colp, zn, zd)
        if r < ROUNDS - 1:
            h, t, u, mm = _tc_step(h, num, den, wmax, wiht, whht, bih2, bhh2,
                                   wlt, wrt, ba12)
        else:
            out = _tc_final(h, num, den, wmax, wiht, whht, bih2, bhh2,
                            wot, bo2)
    return out


# speedup vs baseline: 3.5388x; 1.0164x over previous
import functools

import jax
import jax.numpy as jnp
from jax import lax
from jax.experimental import pallas as pl
from jax.experimental.pallas import tpu as pltpu
from jax.experimental.pallas import tpu_sc as plsc

N = 10000
E = 320000
H = 128
ROUNDS = 5

NC, NS, L = 2, 16, 16
NW = NC * NS
CH = 32
SB = 64
NSC = 5
NCHUNK = NSC * SB
EPW = NCHUNK * CH
EPAD = NW * EPW
RPT = N // NS

_MESH = plsc.VectorSubcoreMesh(core_axis_name="c", subcore_axis_name="s")


def _hsum(v):
    return plsc.cumsum(v)[15]


def _hmax(v):
    return plsc.cummax(v)[15]


def _minmax(a, b, i, mm_ref):
    new = jnp.concatenate(
        [jnp.max(a, axis=0, keepdims=True), jnp.min(a, axis=0, keepdims=True),
         jnp.max(b, axis=0, keepdims=True), jnp.min(b, axis=0, keepdims=True)],
        axis=0)
    cur = mm_ref[...]
    comb = jnp.concatenate(
        [jnp.maximum(cur[0:1], new[0:1]), jnp.minimum(cur[1:2], new[1:2]),
         jnp.maximum(cur[2:3], new[2:3]), jnp.minimum(cur[3:4], new[3:4])],
        axis=0)
    mm_ref[...] = jnp.where(i == 0, new, comb)


def _pre_body(x_ref, wpt, bp, wlt, wrt, ba1, h_ref, t_ref, u_ref, mm_ref):
    i = pl.program_id(0)
    h = jnp.maximum(jnp.dot(x_ref[...], wpt[...],
                            preferred_element_type=jnp.float32) + bp[...], 0.0)
    h_ref[...] = h
    a = jnp.dot(h, wlt[...], preferred_element_type=jnp.float32)
    b = jnp.dot(h, wrt[...], preferred_element_type=jnp.float32) + ba1[...]
    t_ref[...] = jnp.concatenate([a, h], axis=1)
    u_ref[...] = b
    _minmax(a, b, i, mm_ref)


def _gru(h, num_ref, den_ref, wmax_ref, wiht, whht, bih, bhh):
    c = jnp.max(wmax_ref[...])
    num = num_ref[0] + num_ref[1]
    den = jnp.sum(den_ref[0] + den_ref[1], axis=-1, keepdims=True)
    agg = num / (den + 1e-8 * c)
    gi = jnp.dot(agg, wiht[...], preferred_element_type=jnp.float32) + bih[...]
    gh = jnp.dot(h, whht[...], preferred_element_type=jnp.float32) + bhh[...]
    r = jax.nn.sigmoid(gi[:, 0:H] + gh[:, 0:H])
    z = jax.nn.sigmoid(gi[:, H:2 * H] + gh[:, H:2 * H])
    n = jnp.tanh(gi[:, 2 * H:] + r * gh[:, 2 * H:])
    return (1.0 - z) * n + z * h


def _step_body(h_ref, num_ref, den_ref, wmax_ref, wiht, whht, bih, bhh,
               wlt, wrt, ba1, h_out, t_out, u_out, mm_ref):
    i = pl.program_id(0)
    hn = _gru(h_ref[...], num_ref, den_ref, wmax_ref, wiht, whht, bih, bhh)
    h_out[...] = hn
    a = jnp.dot(hn, wlt[...], preferred_element_type=jnp.float32)
    b = jnp.dot(hn, wrt[...], preferred_element_type=jnp.float32) + ba1[...]
    t_out[...] = jnp.concatenate([a, hn], axis=1)
    u_out[...] = b
    _minmax(a, b, i, mm_ref)


def _final_body(h_ref, num_ref, den_ref, wmax_ref, wiht, whht, bih, bhh,
                wot, bo, out_ref):
    hn = _gru(h_ref[...], num_ref, den_ref, wmax_ref, wiht, whht, bih, bhh)
    out_ref[...] = jnp.dot(hn, wot[...],
                           preferred_element_type=jnp.float32) + bo[...]


_BLK = 1000
_GRID = N // _BLK


def _row_spec():
    return pl.BlockSpec((_BLK, H), lambda i: (i, 0))


def _row_spec2():
    return pl.BlockSpec((_BLK, 2 * H), lambda i: (i, 0))


def _whole(shape):
    return pl.BlockSpec(shape, lambda i: tuple(0 for _ in shape))


def _tc_pre(x, wpt, bp, wlt, wrt, ba1):
    return pl.pallas_call(
        _pre_body,
        grid=(_GRID,),
        in_specs=[_row_spec(), _whole((H, H)), _whole((1, H)),
                  _whole((H, H)), _whole((H, H)), _whole((1, H))],
        out_specs=[_row_spec(), _row_spec2(), _row_spec(), _whole((4, H))],
        out_shape=[jax.ShapeDtypeStruct((N, H), jnp.float32),
                   jax.ShapeDtypeStruct((N, 2 * H), jnp.float32),
                   jax.ShapeDtypeStruct((N, H), jnp.float32),
                   jax.ShapeDtypeStruct((4, H), jnp.float32)],
    )(x, wpt, bp, wlt, wrt, ba1)


def _agg_specs():
    return [pl.BlockSpec((2, _BLK, H), lambda i: (0, i, 0)),
            pl.BlockSpec((2, _BLK, L), lambda i: (0, i, 0)),
            _whole((NW, L))]


def _tc_step(h, num, den, wmax, wiht, whht, bih, bhh, wlt, wrt, ba1):
    return pl.pallas_call(
        _step_body,
        grid=(_GRID,),
        in_specs=[_row_spec()] + _agg_specs() +
                 [_whole((H, 3 * H)), _whole((H, 3 * H)), _whole((1, 3 * H)),
                  _whole((1, 3 * H)), _whole((H, H)), _whole((H, H)),
                  _whole((1, H))],
        out_specs=[_row_spec(), _row_spec2(), _row_spec(), _whole((4, H))],
        out_shape=[jax.ShapeDtypeStruct((N, H), jnp.float32),
                   jax.ShapeDtypeStruct((N, 2 * H), jnp.float32),
                   jax.ShapeDtypeStruct((N, H), jnp.float32),
                   jax.ShapeDtypeStruct((4, H), jnp.float32)],
    )(h, num, den, wmax, wiht, whht, bih, bhh, wlt, wrt, ba1)


def _tc_final(h, num, den, wmax, wiht, whht, bih, bhh, wot, bo):
    return pl.pallas_call(
        _final_body,
        grid=(_GRID,),
        in_specs=[_row_spec()] + _agg_specs() +
                 [_whole((H, 3 * H)), _whole((H, 3 * H)), _whole((1, 3 * H)),
                  _whole((1, 3 * H)), _whole((H, H)), _whole((1, H))],
        out_specs=[_row_spec()],
        out_shape=[jax.ShapeDtypeStruct((N, H), jnp.float32)],
    )(h, num, den, wmax, wiht, whht, bih, bhh, wot, bo)[0]


@functools.partial(
    pl.kernel,
    out_type=[jax.ShapeDtypeStruct((NC, N, H), jnp.float32),
              jax.ShapeDtypeStruct((NC, N, L), jnp.float32),
              jax.ShapeDtypeStruct((NW, L), jnp.float32)],
    mesh=_MESH,
    compiler_params=pltpu.CompilerParams(needs_layout_passes=False,
                                         use_tc_tiling_on_sc=False),
    scratch_types=[
        pltpu.VMEM((SB, CH), jnp.int32),
        pltpu.VMEM((SB, CH), jnp.int32),
        pltpu.VMEM((CH, 2 * H), jnp.float32),
        pltpu.VMEM((CH, 2 * H), jnp.float32),
        pltpu.VMEM((CH, H), jnp.float32),
        pltpu.VMEM((CH, H), jnp.float32),
        pltpu.VMEM((CH, H), jnp.float32),
        pltpu.VMEM((CH, H), jnp.float32),
        pltpu.VMEM((CH, L), jnp.float32),
        pltpu.VMEM((CH, L), jnp.float32),
        pltpu.VMEM((H,), jnp.float32),
        pltpu.VMEM((4, H), jnp.float32),
        pltpu.VMEM_SHARED((N, H), jnp.float32),
        pltpu.VMEM_SHARED((N, L), jnp.float32),
        pltpu.SemaphoreType.DMA,
        pltpu.SemaphoreType.DMA,
        pltpu.SemaphoreType.DMA,
        pltpu.SemaphoreType.DMA,
        pltpu.SemaphoreType.DMA,
        pltpu.SemaphoreType.DMA,
        pltpu.SemaphoreType.DMA,
        pltpu.SemaphoreType.DMA,
        pltpu.SemaphoreType.DMA,
        pltpu.SemaphoreType.DMA,
        pltpu.SemaphoreType.DMA,
        pltpu.SemaphoreType.DMA,
    ],
)
def _sc_edge(t_hbm, b_hbm, h_hbm, wa2_hbm, mm_hbm, row_hbm, col_hbm,
             zn_hbm, zd_hbm, num_hbm, den_hbm, wmax_hbm,
             row_v, col_v, gt0, gt1, gb0, gb1, gv0, gv1, db0, db1,
             wa2_v, mm_v, acc_num, acc_den,
             st0, su0, st1, su1, sv0, sv1,
             sn0, sd0, sn1, sd1, si0, si1):
    c = lax.axis_index("c")
    sid = lax.axis_index("s")
    wid = sid * NC + c
    pltpu.sync_copy(wa2_hbm, wa2_v)
    pltpu.sync_copy(mm_hbm, mm_v)
    wvecs = [wa2_v[pl.ds(16 * k, 16)] for k in range(8)]
    lanes = lax.iota(jnp.int32, 16)
    masks = [lanes == l for l in range(16)]
    mask0 = masks[0]
    zero16 = jnp.zeros((16,), jnp.float32)

    ub = jnp.zeros((16,), jnp.float32)
    for k in range(8):
        wk = wvecs[k]
        hi_ab = jnp.maximum(mm_v[0, pl.ds(16 * k, 16)]
                            + mm_v[2, pl.ds(16 * k, 16)], 0.0)
        lo_ab = jnp.maximum(mm_v[1, pl.ds(16 * k, 16)]
                            + mm_v[3, pl.ds(16 * k, 16)], 0.0)
        ub = ub + jnp.where(wk >= 0.0, wk * hi_ab, wk * lo_ab)
    mhat = _hsum(ub)

    tbase = sid * RPT
    pltpu.sync_copy(zn_hbm, acc_num.at[pl.ds(tbase, RPT)])
    pltpu.sync_copy(zd_hbm, acc_den.at[pl.ds(tbase, RPT)])
    plsc.subcore_barrier()

    slots = ((gt0, gb0, gv0, db0, st0, su0, sv0, sn0, sd0),
             (gt1, gb1, gv1, db1, st1, su1, sv1, sn1, sd1))

    def issue_gather_t(jj, slot):
        gt, gb, _, _, st, su, _, _, _ = slots[slot]
        pltpu.async_copy(t_hbm.at[row_v.at[jj]], gt, st)
        pltpu.async_copy(b_hbm.at[col_v.at[jj]], gb, su)

    def issue_gather_u(jj, slot):
        _, _, gv, _, _, _, sv, _, _ = slots[slot]
        pltpu.async_copy(h_hbm.at[col_v.at[jj]], gv, sv)

    def wait_gather(slot):
        gt, gb, gv, _, st, su, sv, _, _ = slots[slot]
        pltpu.make_async_copy(t_hbm.at[row_v.at[0]], gt, st).wait()
        pltpu.make_async_copy(b_hbm.at[col_v.at[0]], gb, su).wait()
        pltpu.make_async_copy(h_hbm.at[col_v.at[0]], gv, sv).wait()

    def issue_scatter(jj, slot):
        _, _, gv, db, _, _, _, sn, sd = slots[slot]
        pltpu.async_copy(gv, acc_num.at[row_v.at[jj]], sn, add=True)
        pltpu.async_copy(db, acc_den.at[row_v.at[jj]], sd, add=True)

    def wait_scatter(slot):
        _, _, gv, db, _, _, _, sn, sd = slots[slot]
        pltpu.make_async_copy(gv, acc_num.at[row_v.at[0]], sn).wait()
        pltpu.make_async_copy(db, acc_den.at[row_v.at[0]], sd).wait()

    def compute(j, slot, wmax):
        gt, gb, gv, db, _, _, _, _, _ = slots[slot]

        def group(g, wmax):
            svec = jnp.zeros((16,), jnp.float32)
            for l in range(16):
                e = g * 16 + l
                acc = jnp.zeros((16,), jnp.float32)
                for k in range(8):
                    va = gt[e, pl.ds(16 * k, 16)]
                    vb = gb[e, pl.ds(16 * k, 16)]
                    acc = acc + jnp.maximum(va + vb, 0.0) * wvecs[k]
                svec = jnp.where(masks[l], _hsum(acc), svec)
            base = wid * EPW + j * CH + g * 16
            wv = jnp.exp(svec - mhat)
            wv = jnp.where(lanes + base < E, wv, 0.0)
            for l in range(16):
                e = g * 16 + l
                ws = wv[l]
                db[e, pl.ds(0, 16)] = jnp.where(mask0, ws, zero16)
                for k in range(8):
                    d = jnp.abs(gt[e, pl.ds(H + 16 * k, 16)]
                                - gv[e, pl.ds(16 * k, 16)])
                    gv[e, pl.ds(16 * k, 16)] = d * ws
            return jnp.maximum(wmax, _hmax(wv))

        return lax.fori_loop(0, CH // 16, group, wmax)

    def superchunk(sc, wmax):
        ci = pltpu.async_copy(row_hbm.at[wid, pl.ds(sc * SB, SB)], row_v, si0)
        cj = pltpu.async_copy(col_hbm.at[wid, pl.ds(sc * SB, SB)], col_v, si1)
        ci.wait()
        cj.wait()
        issue_gather_t(0, 0)
        issue_gather_u(0, 0)
        issue_gather_t(1, 1)
        issue_gather_u(1, 1)

        def pair(p, wmax):
            jj = 2 * p
            j = sc * SB + jj
            wait_gather(0)
            wmax = compute(j, 0, wmax)
            issue_scatter(jj, 0)

            @pl.when(jj + 2 < SB)
            def _():
                issue_gather_t(jj + 2, 0)

            wait_gather(1)
            wmax = compute(j + 1, 1, wmax)
            issue_scatter(jj + 1, 1)

            @pl.when(jj + 3 < SB)
            def _():
                issue_gather_t(jj + 3, 1)

            wait_scatter(0)

            @pl.when(jj + 2 < SB)
            def _():
                issue_gather_u(jj + 2, 0)

            wait_scatter(1)

            @pl.when(jj + 3 < SB)
            def _():
                issue_gather_u(jj + 3, 1)

            return wmax

        return lax.fori_loop(0, SB // 2, pair, wmax)

    wmax = lax.fori_loop(0, NSC, superchunk, jnp.float32(0.0))
    plsc.subcore_barrier()
    pltpu.sync_copy(acc_num.at[pl.ds(tbase, RPT)],
                    num_hbm.at[c, pl.ds(tbase, RPT)])
    pltpu.sync_copy(acc_den.at[pl.ds(tbase, RPT)],
                    den_hbm.at[c, pl.ds(tbase, RPT)])
    db0[0, pl.ds(0, 16)] = jnp.broadcast_to(wmax, (16,))
    pltpu.sync_copy(db0.at[0, pl.ds(0, 16)], wmax_hbm.at[wid])


def kernel(x, edge_index, Wp, bp, Wa1, ba1, Wa2, ba2, Wih, Whh, bih, bhh,
           Wo, bo):
    del ba2
    row = edge_index[0].astype(jnp.int32)
    col = edge_index[1].astype(jnp.int32)
    pad = jnp.zeros((EPAD - E,), jnp.int32)
    rowp = jnp.concatenate([row, pad]).reshape(NW, NCHUNK, CH)
    colp = jnp.concatenate([col, pad]).reshape(NW, NCHUNK, CH)
    zn = jnp.zeros((RPT, H), jnp.float32)
    zd = jnp.zeros((RPT, L), jnp.float32)

    wpt = Wp.T
    wlt = Wa1[:, :H].T
    wrt = Wa1[:, H:].T
    wa2v = Wa2.reshape(H)
    wiht = Wih.T
    whht = Whh.T
    wot = Wo.T
    bp2 = bp.reshape(1, H)
    ba12 = ba1.reshape(1, H)
    bih2 = bih.reshape(1, 3 * H)
    bhh2 = bhh.reshape(1, 3 * H)
    bo2 = bo.reshape(1, H)

    h, t, u, mm = _tc_pre(x, wpt, bp2, wlt, wrt, ba12)
    for r in range(ROUNDS):
        num, den, wmax = _sc_edge(t, u, h, wa2v, mm, rowp, colp, zn, zd)
        if r < ROUNDS - 1:
            h, t, u, mm = _tc_step(h, num, den, wmax, wiht, whht, bih2, bhh2,
                                   wlt, wrt, ba12)
        else:
            out = _tc_final(h, num, den, wmax, wiht, whht, bih2, bhh2,
                            wot, bo2)
    return out
